# Initial kernel scaffold; baseline (speedup 1.0000x reference)
import functools

import jax
import jax.numpy as jnp
from jax import lax
from jax.experimental import pallas as pl
from jax.experimental.pallas import tpu as pltpu
from jax.experimental.pallas import tpu_sc as plsc

D = 128
L = 16
NVEC = D // L
NC, NS = 2, 16
NW = NC * NS
B, SEQ = 1024, 200
N_ROWS = B * SEQ
ROWS_PER_W = N_ROWS // NW
G = 128
NG = ROWS_PER_W // G
NBUF = 4
EPS = 1e-12
INV_D = 1.0 / D


def _rsqrt_vec(v):
    i = plsc.bitcast(v, jnp.int32)
    i = jnp.int32(0x5F3759DF) - (i >> 1)
    y = plsc.bitcast(i, jnp.float32)
    vh = v * 0.5
    for _ in range(3):
        y = y * (1.5 - vh * y * y)
    return y


def _ln_group(buf, gs, bs):
    @pl.loop(0, G, unroll=2)
    def _(r):
        xs = [buf[r, pl.ds(L * k, L)] for k in range(NVEC)]
        s = xs[0]
        q = xs[0] * xs[0]
        for k in range(1, NVEC):
            s = s + xs[k]
            q = q + xs[k] * xs[k]
        mean = jnp.sum(s) * INV_D
        var = jnp.sum(q) * INV_D - mean * mean
        var = jnp.maximum(var, 0.0) + EPS
        inv = _rsqrt_vec(jnp.broadcast_to(var, (L,)))
        for k in range(NVEC):
            buf[r, pl.ds(L * k, L)] = (xs[k] - mean) * inv * gs[k] + bs[k]


def _body(seq_r, table_r, gamma_r, beta_r, out_r,
          idx_v, b0, b1, b2, b3, gamma_v, beta_v,
          gs0, gs1, gs2, gs3, os0, os1, os2, os3):
    bufs = (b0, b1, b2, b3)
    gsems = (gs0, gs1, gs2, gs3)
    osems = (os0, os1, os2, os3)

    wid = lax.axis_index("s") * NC + lax.axis_index("c")
    base = wid * ROWS_PER_W

    pltpu.sync_copy(seq_r.at[pl.ds(wid * NG, NG)], idx_v)
    pltpu.sync_copy(gamma_r, gamma_v)
    pltpu.sync_copy(beta_r, beta_v)
    gs = [gamma_v[pl.ds(L * k, L)] for k in range(NVEC)]
    bs = [beta_v[pl.ds(L * k, L)] for k in range(NVEC)]

    def start_gather(g, b):
        pltpu.async_copy(table_r.at[idx_v.at[g]], bufs[b], gsems[b])

    def wait_gather(g, b):
        pltpu.make_async_copy(table_r.at[idx_v.at[g]], bufs[b], gsems[b]).wait()

    def start_out(g, b):
        pltpu.async_copy(bufs[b], out_r.at[pl.ds(base + g * G, G)], osems[b])

    def wait_out(g, b):
        pltpu.make_async_copy(bufs[b], out_r.at[pl.ds(base + g * G, G)], osems[b]).wait()

    for b in range(NBUF - 1):
        start_gather(b, b)

    @pl.loop(0, NG // NBUF)
    def _(gq):
        for s in range(NBUF):
            g = gq * NBUF + s
            wait_gather(g, s)
            nxt = (s + NBUF - 1) % NBUF

            @pl.when(g + NBUF - 1 < NG)
            def _():
                @pl.when(g >= 1)
                def _():
                    wait_out(g - 1, nxt)
                start_gather(g + NBUF - 1, nxt)

            _ln_group(bufs[s], gs, bs)
            start_out(g, s)

    for s in range(NG % NBUF):
        g = (NG // NBUF) * NBUF + s
        wait_gather(g, s)
        _ln_group(bufs[s], gs, bs)
        start_out(g, s)

    for g in range(NG - NBUF, NG):
        wait_out(g, g % NBUF)


@jax.jit
def _emb_ln(seq2, table, gamma, beta):
    mesh = plsc.VectorSubcoreMesh(core_axis_name="c", subcore_axis_name="s")
    f = pl.kernel(
        _body,
        out_type=jax.ShapeDtypeStruct((N_ROWS, D), jnp.float32),
        mesh=mesh,
        scratch_types=[
            pltpu.VMEM((NG, G), jnp.int32),
            pltpu.VMEM((G, D), jnp.float32),
            pltpu.VMEM((G, D), jnp.float32),
            pltpu.VMEM((G, D), jnp.float32),
            pltpu.VMEM((G, D), jnp.float32),
            pltpu.VMEM((D,), jnp.float32),
            pltpu.VMEM((D,), jnp.float32),
            pltpu.SemaphoreType.DMA,
            pltpu.SemaphoreType.DMA,
            pltpu.SemaphoreType.DMA,
            pltpu.SemaphoreType.DMA,
            pltpu.SemaphoreType.DMA,
            pltpu.SemaphoreType.DMA,
            pltpu.SemaphoreType.DMA,
            pltpu.SemaphoreType.DMA,
        ],
    )
    return f(seq2, table, gamma, beta)


def kernel(seq, table, gamma, beta):
    seq2 = seq.reshape(N_ROWS // G, G).astype(jnp.int32)
    out = _emb_ln(seq2, table, gamma, beta)
    return out.reshape(B, SEQ, D)


# speedup vs baseline: 4.1587x; 4.1587x over previous
import functools

import jax
import jax.numpy as jnp
from jax import lax
from jax.experimental import pallas as pl
from jax.experimental.pallas import tpu as pltpu
from jax.experimental.pallas import tpu_sc as plsc

D = 128
L = 16
NVEC = D // L
NC, NS = 2, 16
NW = NC * NS
B, SEQ = 1024, 200
N_ROWS = B * SEQ
ROWS_PER_W = N_ROWS // NW
G = 128
NG = ROWS_PER_W // G
NBUF = 4
EPS = 1e-12
INV_D = 1.0 / D


def _rsqrt_vec(v):
    i = plsc.bitcast(v, jnp.int32)
    i = jnp.int32(0x5F3759DF) - (i >> 1)
    y = plsc.bitcast(i, jnp.float32)
    vh = v * 0.5
    for _ in range(3):
        y = y * (1.5 - vh * y * y)
    return y


def _ln_group(buf, gs, bs):
    @pl.loop(0, G, unroll=2)
    def _(r):
        xs = [buf[r, pl.ds(L * k, L)] for k in range(NVEC)]
        s = xs[0]
        q = xs[0] * xs[0]
        for k in range(1, NVEC):
            s = s + xs[k]
            q = q + xs[k] * xs[k]
        mean = jnp.sum(s) * INV_D
        var = jnp.sum(q) * INV_D - mean * mean
        var = jnp.maximum(var, 0.0) + EPS
        inv = _rsqrt_vec(jnp.broadcast_to(var, (L,)))
        for k in range(NVEC):
            buf[r, pl.ds(L * k, L)] = (xs[k] - mean) * inv * gs[k] + bs[k]


def _body(seq_r, table_r, gamma_r, beta_r, out_r,
          idx_v, b0, b1, b2, b3, gamma_v, beta_v,
          gs0, gs1, gs2, gs3, os0, os1, os2, os3):
    bufs = (b0, b1, b2, b3)
    gsems = (gs0, gs1, gs2, gs3)
    osems = (os0, os1, os2, os3)

    wid = lax.axis_index("s") * NC + lax.axis_index("c")
    base = wid * ROWS_PER_W

    pltpu.sync_copy(seq_r.at[pl.ds(pl.multiple_of(base, G), ROWS_PER_W)], idx_v)
    pltpu.sync_copy(gamma_r, gamma_v)
    pltpu.sync_copy(beta_r, beta_v)
    gs = [gamma_v[pl.ds(L * k, L)] for k in range(NVEC)]
    bs = [beta_v[pl.ds(L * k, L)] for k in range(NVEC)]

    def idx_slice(g):
        return idx_v.at[pl.ds(pl.multiple_of(g * G, G), G)]

    def start_gather(g, b):
        pltpu.async_copy(table_r.at[idx_slice(g)], bufs[b], gsems[b])

    def wait_gather(g, b):
        pltpu.make_async_copy(table_r.at[idx_slice(g)], bufs[b], gsems[b]).wait()

    def start_out(g, b):
        pltpu.async_copy(bufs[b], out_r.at[pl.ds(base + g * G, G)], osems[b])

    def wait_out(g, b):
        pltpu.make_async_copy(bufs[b], out_r.at[pl.ds(base + g * G, G)], osems[b]).wait()

    for b in range(NBUF - 1):
        start_gather(b, b)

    @pl.loop(0, NG // NBUF)
    def _(gq):
        for s in range(NBUF):
            g = gq * NBUF + s
            wait_gather(g, s)
            nxt = (s + NBUF - 1) % NBUF

            @pl.when(g + NBUF - 1 < NG)
            def _():
                @pl.when(g >= 1)
                def _():
                    wait_out(g - 1, nxt)
                start_gather(g + NBUF - 1, nxt)

            _ln_group(bufs[s], gs, bs)
            start_out(g, s)

    for s in range(NG % NBUF):
        g = (NG // NBUF) * NBUF + s
        wait_gather(g, s)
        _ln_group(bufs[s], gs, bs)
        start_out(g, s)

    for g in range(NG - NBUF, NG):
        wait_out(g, g % NBUF)


@jax.jit
def _emb_ln(seq2, table, gamma, beta):
    mesh = plsc.VectorSubcoreMesh(core_axis_name="c", subcore_axis_name="s")
    f = pl.kernel(
        _body,
        out_type=jax.ShapeDtypeStruct((N_ROWS, D), jnp.float32),
        mesh=mesh,
        compiler_params=pltpu.CompilerParams(needs_layout_passes=False),
        scratch_types=[
            pltpu.VMEM((ROWS_PER_W,), jnp.int32),
            pltpu.VMEM((G, D), jnp.float32),
            pltpu.VMEM((G, D), jnp.float32),
            pltpu.VMEM((G, D), jnp.float32),
            pltpu.VMEM((G, D), jnp.float32),
            pltpu.VMEM((D,), jnp.float32),
            pltpu.VMEM((D,), jnp.float32),
            pltpu.SemaphoreType.DMA,
            pltpu.SemaphoreType.DMA,
            pltpu.SemaphoreType.DMA,
            pltpu.SemaphoreType.DMA,
            pltpu.SemaphoreType.DMA,
            pltpu.SemaphoreType.DMA,
            pltpu.SemaphoreType.DMA,
            pltpu.SemaphoreType.DMA,
        ],
    )
    return f(seq2, table, gamma, beta)


def kernel(seq, table, gamma, beta):
    seq2 = seq.reshape(N_ROWS).astype(jnp.int32)
    out = _emb_ln(seq2, table, gamma, beta)
    return out.reshape(B, SEQ, D)


# speedup vs baseline: 4.4680x; 1.0744x over previous
import functools

import jax
import jax.numpy as jnp
from jax import lax
from jax.experimental import pallas as pl
from jax.experimental.pallas import tpu as pltpu
from jax.experimental.pallas import tpu_sc as plsc

D = 128
L = 16
NVEC = D // L
NC, NS = 2, 16
NW = NC * NS
B, SEQ = 1024, 200
N_ROWS = B * SEQ
ROWS_PER_W = N_ROWS // NW
G = 128
NG = ROWS_PER_W // G
NBUF = 4
EPS = 1e-12
INV_D = 1.0 / D


def _rsqrt_vec(v):
    i = plsc.bitcast(v, jnp.int32)
    i = jnp.int32(0x5F3759DF) - (i >> 1)
    y = plsc.bitcast(i, jnp.float32)
    vh = v * 0.5
    for _ in range(2):
        y = y * (1.5 - vh * y * y)
    return y


def _ln_group(buf, gs, bs):
    @pl.loop(0, G, unroll=2)
    def _(r):
        xs = [buf[r, pl.ds(L * k, L)] for k in range(NVEC)]
        s = xs[0]
        q = xs[0] * xs[0]
        for k in range(1, NVEC):
            s = s + xs[k]
            q = q + xs[k] * xs[k]
        mean = jnp.sum(s) * INV_D
        var = jnp.sum(q) * INV_D - mean * mean
        var = jnp.maximum(var, 0.0) + EPS
        inv = _rsqrt_vec(jnp.broadcast_to(var, (L,)))
        for k in range(NVEC):
            buf[r, pl.ds(L * k, L)] = (xs[k] - mean) * inv * gs[k] + bs[k]


def _body(seq_r, table_r, gamma_r, beta_r, out_r,
          idx_v, b0, b1, b2, b3, gamma_v, beta_v,
          gs0, gs1, gs2, gs3, os0, os1, os2, os3):
    bufs = (b0, b1, b2, b3)
    gsems = (gs0, gs1, gs2, gs3)
    osems = (os0, os1, os2, os3)

    wid = lax.axis_index("s") * NC + lax.axis_index("c")
    base = wid * ROWS_PER_W

    pltpu.sync_copy(seq_r.at[pl.ds(pl.multiple_of(base, G), ROWS_PER_W)], idx_v)
    pltpu.sync_copy(gamma_r, gamma_v)
    pltpu.sync_copy(beta_r, beta_v)
    gs = [gamma_v[pl.ds(L * k, L)] for k in range(NVEC)]
    bs = [beta_v[pl.ds(L * k, L)] for k in range(NVEC)]

    def idx_slice(g):
        return idx_v.at[pl.ds(pl.multiple_of(g * G, G), G)]

    def start_gather(g, b):
        pltpu.async_copy(table_r.at[idx_slice(g)], bufs[b], gsems[b])

    def wait_gather(g, b):
        pltpu.make_async_copy(table_r.at[idx_slice(g)], bufs[b], gsems[b]).wait()

    def start_out(g, b):
        pltpu.async_copy(bufs[b], out_r.at[pl.ds(base + g * G, G)], osems[b])

    def wait_out(g, b):
        pltpu.make_async_copy(bufs[b], out_r.at[pl.ds(base + g * G, G)], osems[b]).wait()

    for b in range(NBUF - 1):
        start_gather(b, b)

    @pl.loop(0, NG // NBUF)
    def _(gq):
        for s in range(NBUF):
            g = gq * NBUF + s
            wait_gather(g, s)
            nxt = (s + NBUF - 1) % NBUF

            @pl.when(g + NBUF - 1 < NG)
            def _():
                @pl.when(g >= 1)
                def _():
                    wait_out(g - 1, nxt)
                start_gather(g + NBUF - 1, nxt)

            _ln_group(bufs[s], gs, bs)
            start_out(g, s)

    for s in range(NG % NBUF):
        g = (NG // NBUF) * NBUF + s
        wait_gather(g, s)
        _ln_group(bufs[s], gs, bs)
        start_out(g, s)

    for g in range(NG - NBUF, NG):
        wait_out(g, g % NBUF)


@jax.jit
def _emb_ln(seq2, table, gamma, beta):
    mesh = plsc.VectorSubcoreMesh(core_axis_name="c", subcore_axis_name="s")
    f = pl.kernel(
        _body,
        out_type=jax.ShapeDtypeStruct((N_ROWS, D), jnp.float32),
        mesh=mesh,
        compiler_params=pltpu.CompilerParams(needs_layout_passes=False),
        scratch_types=[
            pltpu.VMEM((ROWS_PER_W,), jnp.int32),
            pltpu.VMEM((G, D), jnp.float32),
            pltpu.VMEM((G, D), jnp.float32),
            pltpu.VMEM((G, D), jnp.float32),
            pltpu.VMEM((G, D), jnp.float32),
            pltpu.VMEM((D,), jnp.float32),
            pltpu.VMEM((D,), jnp.float32),
            pltpu.SemaphoreType.DMA,
            pltpu.SemaphoreType.DMA,
            pltpu.SemaphoreType.DMA,
            pltpu.SemaphoreType.DMA,
            pltpu.SemaphoreType.DMA,
            pltpu.SemaphoreType.DMA,
            pltpu.SemaphoreType.DMA,
            pltpu.SemaphoreType.DMA,
        ],
    )
    return f(seq2, table, gamma, beta)


def kernel(seq, table, gamma, beta):
    seq2 = seq.reshape(N_ROWS).astype(jnp.int32)
    out = _emb_ln(seq2, table, gamma, beta)
    return out.reshape(B, SEQ, D)
